# Initial kernel scaffold; baseline (speedup 1.0000x reference)
import jax
import jax.numpy as jnp
from jax.experimental import pallas as pl


def kernel(P1, X1, P2, H2, C2, Wi, bi, Wf, bf, Wo, bo, Wn, bn_, Wold, bold):
    raise NotImplementedError("write your pallas kernel here")


# speedup vs baseline: 15.0349x; 15.0349x over previous
import functools

import jax
import jax.numpy as jnp
from jax.experimental import pallas as pl
from jax.experimental.pallas import tpu as pltpu

RADIUS = 0.2
K = 16
N = 1024
QB = 256
CH = 640


def _prep_body(h2t_ref, c2t_ref, x1t_ref, p1_ref, p2_ref,
               wst_ref, woldst_ref, wxt_ref, wd4t_ref, wdoldt_ref,
               b4_ref, bold_ref, g_ref, cst_ref):
    h2t = h2t_ref[...]
    p2 = p2_ref[...]
    g4 = jnp.dot(h2t, wst_ref[...], preferred_element_type=jnp.float32)
    g4 = g4 + jnp.dot(p2, wd4t_ref[...], preferred_element_type=jnp.float32)
    gold = jnp.dot(c2t_ref[...], woldst_ref[...], preferred_element_type=jnp.float32)
    gold = gold + jnp.dot(p2, wdoldt_ref[...], preferred_element_type=jnp.float32)
    g_ref[...] = jnp.concatenate([g4, gold], axis=1).astype(jnp.bfloat16)
    p1 = p1_ref[...]
    c4 = jnp.dot(x1t_ref[...], wxt_ref[...], preferred_element_type=jnp.float32)
    c4 = c4 - jnp.dot(p1, wd4t_ref[...], preferred_element_type=jnp.float32)
    c4 = c4 + b4_ref[...]
    cold = bold_ref[...] - jnp.dot(p1, wdoldt_ref[...], preferred_element_type=jnp.float32)
    cst_ref[...] = jnp.concatenate([c4, cold], axis=1)


def _main_body(g_ref, cst_ref, p1_ref, p2t_ref, h1_ref, c1_ref, tri_ref):
    b = pl.program_id(0)
    q = pl.program_id(1)

    @pl.when(jnp.logical_and(b == 0, q == 0))
    def _build_tri():
        rows = jax.lax.broadcasted_iota(jnp.int32, (N, N), 0)
        cols = jax.lax.broadcasted_iota(jnp.int32, (N, N), 1)
        tri_ref[...] = jnp.where(rows <= cols, 1.0, 0.0).astype(jnp.bfloat16)

    p1 = p1_ref[...]
    p2t = p2t_ref[...]
    d2 = jnp.zeros((QB, N), jnp.float32)
    for c in range(3):
        diff = p1[:, c:c + 1] - p2t[c:c + 1, :]
        d2 = d2 + diff * diff
    mask = d2 < jnp.float32(RADIUS * RADIUS)
    maskbf = jnp.where(mask, 1.0, 0.0).astype(jnp.bfloat16)
    cum = jnp.dot(maskbf, tri_ref[...], preferred_element_type=jnp.float32)
    count = jnp.sum(jnp.where(mask, 1.0, 0.0), axis=1, keepdims=True)

    gbf = g_ref[...]
    neg = jnp.float32(-jnp.inf)
    macc = jnp.full((QB, CH), neg, jnp.float32)
    for k in range(1, K + 1):
        oh = jnp.where(jnp.logical_and(mask, cum == k), 1.0, 0.0).astype(jnp.bfloat16)
        mk = jnp.dot(oh, gbf, preferred_element_type=jnp.float32)
        macc = jnp.maximum(macc, jnp.where(count >= k, mk, neg))
    g0 = gbf[0:1, :].astype(jnp.float32)
    macc = jnp.where(count == 0, g0, macc)

    a = jax.nn.relu(macc + cst_ref[...])
    gi = jax.nn.sigmoid(a[:, 0:128])
    gf = jax.nn.sigmoid(a[:, 128:256])
    go = jax.nn.sigmoid(a[:, 256:384])
    cn = jnp.tanh(a[:, 384:512])
    co = a[:, 512:640]
    c1 = gf * co + gi * cn
    h1_ref[...] = go * jnp.tanh(c1)
    c1_ref[...] = c1


@jax.jit
def kernel(P1, X1, P2, H2, C2, Wi, bi, Wf, bf, Wo, bo, Wn, bn_, Wold, bold):
    B = P1.shape[0]
    W_ST = jnp.concatenate([Wi[:, :128], Wf[:, :128], Wo[:, :128], Wn[:, :128]], 0).T
    W_XT = jnp.concatenate([Wi[:, 128:256], Wf[:, 128:256], Wo[:, 128:256], Wn[:, 128:256]], 0).T
    W_D4T = jnp.concatenate([Wi[:, 256:], Wf[:, 256:], Wo[:, 256:], Wn[:, 256:]], 0).T
    WoldST = Wold[:, :128].T
    W_DoldT = Wold[:, 128:].T
    b4 = jnp.concatenate([bi, bf, bo, bn_], 0)[None, :]
    bold2 = bold[None, :]
    H2T = jnp.transpose(H2, (0, 2, 1))
    C2T = jnp.transpose(C2, (0, 2, 1))
    X1T = jnp.transpose(X1, (0, 2, 1))
    P2T = jnp.transpose(P2, (0, 2, 1))

    g, cst = pl.pallas_call(
        _prep_body,
        grid=(B,),
        in_specs=[
            pl.BlockSpec((None, N, 128), lambda b: (b, 0, 0)),
            pl.BlockSpec((None, N, 128), lambda b: (b, 0, 0)),
            pl.BlockSpec((None, N, 128), lambda b: (b, 0, 0)),
            pl.BlockSpec((None, N, 3), lambda b: (b, 0, 0)),
            pl.BlockSpec((None, N, 3), lambda b: (b, 0, 0)),
            pl.BlockSpec((128, 512), lambda b: (0, 0)),
            pl.BlockSpec((128, 128), lambda b: (0, 0)),
            pl.BlockSpec((128, 512), lambda b: (0, 0)),
            pl.BlockSpec((3, 512), lambda b: (0, 0)),
            pl.BlockSpec((3, 128), lambda b: (0, 0)),
            pl.BlockSpec((1, 512), lambda b: (0, 0)),
            pl.BlockSpec((1, 128), lambda b: (0, 0)),
        ],
        out_specs=[
            pl.BlockSpec((None, N, CH), lambda b: (b, 0, 0)),
            pl.BlockSpec((None, N, CH), lambda b: (b, 0, 0)),
        ],
        out_shape=[
            jax.ShapeDtypeStruct((B, N, CH), jnp.bfloat16),
            jax.ShapeDtypeStruct((B, N, CH), jnp.float32),
        ],
    )(H2T, C2T, X1T, P1, P2, W_ST, WoldST, W_XT, W_D4T, W_DoldT, b4, bold2)

    h1t, c1t = pl.pallas_call(
        _main_body,
        grid=(B, N // QB),
        in_specs=[
            pl.BlockSpec((None, N, CH), lambda b, q: (b, 0, 0)),
            pl.BlockSpec((None, QB, CH), lambda b, q: (b, q, 0)),
            pl.BlockSpec((None, QB, 3), lambda b, q: (b, q, 0)),
            pl.BlockSpec((None, 3, N), lambda b, q: (b, 0, 0)),
        ],
        out_specs=[
            pl.BlockSpec((None, QB, 128), lambda b, q: (b, q, 0)),
            pl.BlockSpec((None, QB, 128), lambda b, q: (b, q, 0)),
        ],
        out_shape=[
            jax.ShapeDtypeStruct((B, N, 128), jnp.float32),
            jax.ShapeDtypeStruct((B, N, 128), jnp.float32),
        ],
        scratch_shapes=[pltpu.VMEM((N, N), jnp.bfloat16)],
    )(g, cst, P1, P2T)

    H1 = jnp.transpose(h1t, (0, 2, 1))
    C1 = jnp.transpose(c1t, (0, 2, 1))
    return (P1, H1, C1)
